# baseline (speedup 1.0000x reference)
import dataclasses
import functools

import jax
import jax.numpy as jnp
from jax import lax
from jax.experimental import pallas as pl
from jax.experimental.pallas import tpu as pltpu
from jax.experimental.pallas import tpu_sc as plsc

BATCH = 16384
DIM = 64
LANES = 16
NUM_CORES = 2
NUM_SUBCORES = 16
NUM_WORKERS = NUM_CORES * NUM_SUBCORES
ROWS_PER_WORKER = BATCH // NUM_WORKERS
GROUPS = ROWS_PER_WORKER // LANES

_mesh = plsc.VectorSubcoreMesh(core_axis_name="c", subcore_axis_name="s")

_cp = pltpu.CompilerParams(needs_layout_passes=False,
                           use_tc_tiling_on_sc=False)


@functools.partial(
    pl.kernel,
    out_type=jax.ShapeDtypeStruct((BATCH,), jnp.float32),
    mesh=_mesh,
    compiler_params=_cp,
    scratch_types=[
        pltpu.VMEM((ROWS_PER_WORKER,), jnp.int32),
        pltpu.VMEM((ROWS_PER_WORKER,), jnp.int32),
        pltpu.VMEM((ROWS_PER_WORKER,), jnp.int32),
        pltpu.VMEM((ROWS_PER_WORKER, DIM), jnp.float32),
        pltpu.VMEM((ROWS_PER_WORKER, DIM), jnp.float32),
        pltpu.VMEM((ROWS_PER_WORKER, DIM), jnp.float32),
        pltpu.VMEM((ROWS_PER_WORKER,), jnp.float32),
        pltpu.SemaphoreType.DMA,
        pltpu.SemaphoreType.DMA,
        pltpu.SemaphoreType.DMA,
    ],
)
def _transe_sc(hidx_hbm, ridx_hbm, tidx_hbm, ent_hbm, rel_hbm, out_hbm,
               hi_v, ri_v, ti_v, h_v, r_v, t_v, o_v, sem_h, sem_r, sem_t):
    wid = lax.axis_index("s") * NUM_CORES + lax.axis_index("c")
    base = wid * ROWS_PER_WORKER

    pltpu.sync_copy(hidx_hbm.at[pl.ds(base, ROWS_PER_WORKER)], hi_v)
    pltpu.sync_copy(ridx_hbm.at[pl.ds(base, ROWS_PER_WORKER)], ri_v)
    pltpu.sync_copy(tidx_hbm.at[pl.ds(base, ROWS_PER_WORKER)], ti_v)

    ch = pltpu.async_copy(ent_hbm.at[hi_v], h_v, sem_h)
    cr = pltpu.async_copy(rel_hbm.at[ri_v], r_v, sem_r)
    ct = pltpu.async_copy(ent_hbm.at[ti_v], t_v, sem_t)
    ch.wait()
    cr.wait()
    ct.wait()

    @pl.loop(0, GROUPS)
    def _group(g):
        rows = g * LANES + lax.iota(jnp.int32, LANES)

        def body(d, acc):
            cols = jnp.full((LANES,), d, jnp.int32)
            h = plsc.load_gather(h_v, [rows, cols])
            r = plsc.load_gather(r_v, [rows, cols])
            t = plsc.load_gather(t_v, [rows, cols])
            return acc + jnp.abs(h + r - t)

        acc = lax.fori_loop(0, DIM, body, jnp.zeros((LANES,), jnp.float32))
        o_v[pl.ds(g * LANES, LANES)] = acc

    pltpu.sync_copy(o_v, out_hbm.at[pl.ds(base, ROWS_PER_WORKER)])


def kernel(sample, entity_embedding, relation_embedding):
    h_idx = sample[:, 0]
    r_idx = sample[:, 1]
    t_idx = sample[:, 2]
    scores = _transe_sc(h_idx, r_idx, t_idx, entity_embedding,
                        relation_embedding)
    return scores.reshape(BATCH, 1)
